# baseline (speedup 1.0000x reference)
import functools

import jax
import jax.numpy as jnp
from jax import lax
from jax.experimental import pallas as pl
from jax.experimental.pallas import tpu as pltpu
from jax.experimental.pallas import tpu_sc as plsc

N = 10000
E = 320000
D = 128
NC, NS = 2, 16
NW = NC * NS
CHUNK = 128
EPW = 10240
E_PAD = EPW * NW
CPW = EPW // CHUNK
STAGES = 2
CPS = CPW // STAGES
NPAD = 10240
ZR = NPAD // NS
ZB = 32

_mesh = plsc.VectorSubcoreMesh(
    core_axis_name="c", subcore_axis_name="s", num_cores=NC, num_subcores=NS
)


@functools.partial(
    pl.kernel,
    out_type=jax.ShapeDtypeStruct((NC, NPAD, D), jnp.float32),
    mesh=_mesh,
    scratch_types=[
        pltpu.VMEM((CPS, CHUNK), jnp.int32),
        pltpu.VMEM((CPS, CHUNK), jnp.int32),
        pltpu.VMEM((CHUNK, D), jnp.float32),
        pltpu.VMEM((CHUNK, D), jnp.float32),
        pltpu.VMEM_SHARED((NPAD, D), jnp.float32),
        pltpu.SemaphoreType.DMA,
        pltpu.SemaphoreType.DMA,
    ],
)
def _sc_aggregate(x_hbm, src_hbm, dst_hbm, zeros_hbm, out_hbm,
                  src_v, dst_v, rows0, rows1, acc, sem0, sem1):
    c = lax.axis_index("c")
    s = lax.axis_index("s")
    wid = c * NS + s
    wrow = wid * CPW
    rows = [rows0, rows1]
    sems = [sem0, sem1]
    pltpu.sync_copy(src_hbm.at[pl.ds(wrow, CPS)], src_v)
    pltpu.sync_copy(dst_hbm.at[pl.ds(wrow, CPS)], dst_v)
    cps = [
        pltpu.async_copy(x_hbm.at[src_v.at[j]], rows[j], sems[j])
        for j in range(2)
    ]
    pltpu.sync_copy(zeros_hbm.at[pl.ds(s * ZR, ZR)], acc.at[pl.ds(s * ZR, ZR)])
    plsc.subcore_barrier()

    for stage in range(STAGES):
        if stage > 0:
            srow = wrow + stage * CPS
            pltpu.sync_copy(src_hbm.at[pl.ds(srow, CPS)], src_v)
            pltpu.sync_copy(dst_hbm.at[pl.ds(srow, CPS)], dst_v)
            cps = [
                pltpu.async_copy(x_hbm.at[src_v.at[j]], rows[j], sems[j])
                for j in range(2)
            ]
        for j in range(CPS):
            b = j % 2
            cps[b].wait()
            pltpu.sync_copy(rows[b], acc.at[dst_v.at[j]], add=True)
            if j + 2 < CPS:
                cps[b] = pltpu.async_copy(
                    x_hbm.at[src_v.at[j + 2]], rows[b], sems[b]
                )
    plsc.subcore_barrier()
    pltpu.sync_copy(acc.at[pl.ds(s * ZR, ZR)], out_hbm.at[c, pl.ds(s * ZR, ZR), :])


def _mlp_body(x_r, p_r, wa_r, ba_r, wb_r, bb_r, o_r):
    pr = p_r[...]
    h = x_r[...] + pr[0] + pr[1]
    t = jnp.dot(h, wa_r[...], preferred_element_type=jnp.float32) + ba_r[...]
    t = jnp.maximum(t, 0.0)
    o = jnp.dot(t, wb_r[...], preferred_element_type=jnp.float32) + bb_r[...]
    o_r[...] = jnp.maximum(o, 0.0)


_BR = 1000


def _mlp(x, p, Wa, ba, Wb, bb):
    return pl.pallas_call(
        _mlp_body,
        grid=(N // _BR,),
        in_specs=[
            pl.BlockSpec((_BR, D), lambda i: (i, 0)),
            pl.BlockSpec((NC, _BR, D), lambda i: (0, i, 0)),
            pl.BlockSpec((D, D), lambda i: (0, 0)),
            pl.BlockSpec((1, D), lambda i: (0, 0)),
            pl.BlockSpec((D, D), lambda i: (0, 0)),
            pl.BlockSpec((1, D), lambda i: (0, 0)),
        ],
        out_specs=pl.BlockSpec((_BR, D), lambda i: (i, 0)),
        out_shape=jax.ShapeDtypeStruct((N, D), jnp.float32),
    )(x, p, Wa, ba, Wb, bb)


def kernel(x, edge_index, W1a, b1a, W1b, b1b, W2a, b2a, W2b, b2b):
    src = edge_index[0].astype(jnp.int32)
    dst = edge_index[1].astype(jnp.int32)
    padw = EPW - E // NW
    pad_src = jnp.broadcast_to((jnp.arange(padw, dtype=jnp.int32) * 41) % N,
                               (NW, padw))
    pad_dst = jnp.broadcast_to(N + jnp.arange(padw, dtype=jnp.int32), (NW, padw))
    src_p = jnp.concatenate([src.reshape(NW, E // NW), pad_src], axis=1)
    dst_p = jnp.concatenate([dst.reshape(NW, E // NW), pad_dst], axis=1)
    src_p = src_p.reshape(E_PAD // CHUNK, CHUNK)
    dst_p = dst_p.reshape(E_PAD // CHUNK, CHUNK)
    zeros = jnp.zeros((NPAD, D), jnp.float32)

    b1a2, b1b2 = b1a.reshape(1, D), b1b.reshape(1, D)
    b2a2, b2b2 = b2a.reshape(1, D), b2b.reshape(1, D)

    p = _sc_aggregate(x, src_p, dst_p, zeros)
    h1 = _mlp(x, p, W1a, b1a2, W1b, b1b2)
    q = _sc_aggregate(h1, src_p, dst_p, zeros)
    h2 = _mlp(h1, q, W2a, b2a2, W2b, b2b2)
    return h2


# speedup vs baseline: 1.0593x; 1.0593x over previous
import functools

import jax
import jax.numpy as jnp
from jax import lax
from jax.experimental import pallas as pl
from jax.experimental.pallas import tpu as pltpu
from jax.experimental.pallas import tpu_sc as plsc

N = 10000
E = 320000
D = 128
DH = D // 2
NC, NS = 2, 16
CHUNK = 128
EPT = 20480
E_PAD = EPT * NS
CPT = EPT // CHUNK
STAGES = 2
CPS = CPT // STAGES
NPAD = 10240
ZR = NPAD // NS
NBUF = 6

_mesh = plsc.VectorSubcoreMesh(
    core_axis_name="c", subcore_axis_name="s", num_cores=NC, num_subcores=NS
)


@functools.partial(
    pl.kernel,
    out_type=jax.ShapeDtypeStruct((NC, NPAD, DH), jnp.float32),
    mesh=_mesh,
    scratch_types=[
        pltpu.VMEM((CPS, CHUNK), jnp.int32),
        pltpu.VMEM((CPS, CHUNK), jnp.int32),
    ]
    + [pltpu.VMEM((CHUNK, DH), jnp.float32)] * NBUF
    + [
        pltpu.VMEM_SHARED((NPAD, DH), jnp.float32),
    ]
    + [pltpu.SemaphoreType.DMA] * NBUF,
    compiler_params=pltpu.CompilerParams(use_tc_tiling_on_sc=False),
)
def _sc_aggregate(x_hbm, src_hbm, dst_hbm, zeros_hbm, out_hbm,
                  src_v, dst_v, *rest):
    rows = list(rest[:NBUF])
    acc = rest[NBUF]
    sems = list(rest[NBUF + 1:])
    c = lax.axis_index("c")
    s = lax.axis_index("s")
    wrow = s * CPT

    def load_stage(stage):
        srow = wrow + stage * CPS
        pltpu.sync_copy(src_hbm.at[c, pl.ds(srow, CPS), :], src_v)
        pltpu.sync_copy(dst_hbm.at[pl.ds(srow, CPS)], dst_v)

    load_stage(0)
    cps = [
        pltpu.async_copy(x_hbm.at[src_v.at[j]], rows[j], sems[j])
        for j in range(NBUF)
    ]
    pltpu.sync_copy(zeros_hbm.at[pl.ds(s * ZR, ZR)], acc.at[pl.ds(s * ZR, ZR)])
    plsc.subcore_barrier()

    for stage in range(STAGES):
        if stage > 0:
            load_stage(stage)
            cps = [
                pltpu.async_copy(x_hbm.at[src_v.at[j]], rows[j], sems[j])
                for j in range(NBUF)
            ]
        for j in range(CPS):
            b = j % NBUF
            cps[b].wait()
            pltpu.sync_copy(rows[b], acc.at[dst_v.at[j]], add=True)
            if j + NBUF < CPS:
                cps[b] = pltpu.async_copy(
                    x_hbm.at[src_v.at[j + NBUF]], rows[b], sems[b]
                )
    plsc.subcore_barrier()
    pltpu.sync_copy(acc.at[pl.ds(s * ZR, ZR)], out_hbm.at[c, pl.ds(s * ZR, ZR), :])


def _mlp_body(x_r, p_r, wa_r, ba_r, wb_r, bb_r, o_r):
    pr = p_r[...]
    h = x_r[...] + jnp.concatenate([pr[0], pr[1]], axis=1)
    t = jnp.dot(h, wa_r[...], preferred_element_type=jnp.float32) + ba_r[...]
    t = jnp.maximum(t, 0.0)
    o = jnp.dot(t, wb_r[...], preferred_element_type=jnp.float32) + bb_r[...]
    o_r[...] = jnp.maximum(o, 0.0)


_BR = 1000


def _mlp(x, p, Wa, ba, Wb, bb):
    return pl.pallas_call(
        _mlp_body,
        grid=(N // _BR,),
        in_specs=[
            pl.BlockSpec((_BR, D), lambda i: (i, 0)),
            pl.BlockSpec((NC, _BR, DH), lambda i: (0, i, 0)),
            pl.BlockSpec((D, D), lambda i: (0, 0)),
            pl.BlockSpec((1, D), lambda i: (0, 0)),
            pl.BlockSpec((D, D), lambda i: (0, 0)),
            pl.BlockSpec((1, D), lambda i: (0, 0)),
        ],
        out_specs=pl.BlockSpec((_BR, D), lambda i: (i, 0)),
        out_shape=jax.ShapeDtypeStruct((N, D), jnp.float32),
    )(x, p, Wa, ba, Wb, bb)


def kernel(x, edge_index, W1a, b1a, W1b, b1b, W2a, b2a, W2b, b2b):
    src = edge_index[0].astype(jnp.int32)
    dst = edge_index[1].astype(jnp.int32)
    padt = EPT - E // NS
    pad_src = jnp.broadcast_to((jnp.arange(padt, dtype=jnp.int32) * 41) % N,
                               (NS, padt))
    pad_dst = jnp.broadcast_to(N + jnp.arange(padt, dtype=jnp.int32) % (NPAD - N),
                               (NS, padt))
    src_p = jnp.concatenate([src.reshape(NS, E // NS), pad_src], axis=1) * 2
    src_p = src_p.reshape(1, E_PAD // CHUNK, CHUNK)
    src_p = jnp.concatenate([src_p, src_p + 1], axis=0)
    dst_p = jnp.concatenate([dst.reshape(NS, E // NS), pad_dst], axis=1)
    dst_p = dst_p.reshape(E_PAD // CHUNK, CHUNK)
    zeros = jnp.zeros((NPAD, DH), jnp.float32)

    b1a2, b1b2 = b1a.reshape(1, D), b1b.reshape(1, D)
    b2a2, b2b2 = b2a.reshape(1, D), b2b.reshape(1, D)

    x2 = x.reshape(2 * N, DH)
    p = _sc_aggregate(x2, src_p, dst_p, zeros)
    h1 = _mlp(x, p, W1a, b1a2, W1b, b1b2)
    q = _sc_aggregate(h1.reshape(2 * N, DH), src_p, dst_p, zeros)
    h2 = _mlp(h1, q, W2a, b2a2, W2b, b2b2)
    return h2


# speedup vs baseline: 1.0781x; 1.0177x over previous
import functools

import jax
import jax.numpy as jnp
from jax import lax
from jax.experimental import pallas as pl
from jax.experimental.pallas import tpu as pltpu
from jax.experimental.pallas import tpu_sc as plsc

N = 10000
E = 320000
D = 128
DH = D // 2
NC, NS = 2, 16
CHUNK = 128
EPT = 20480
E_PAD = EPT * NS
CPT = EPT // CHUNK
NPAD = 10112
ZR = NPAD // NS
NBUF = 6

_mesh = plsc.VectorSubcoreMesh(
    core_axis_name="c", subcore_axis_name="s", num_cores=NC, num_subcores=NS
)


@functools.partial(
    pl.kernel,
    out_type=jax.ShapeDtypeStruct((NC, NPAD, DH), jnp.float32),
    mesh=_mesh,
    scratch_types=[
        pltpu.VMEM((CPT, CHUNK), jnp.int32),
        pltpu.VMEM((CPT, CHUNK), jnp.int32),
    ]
    + [pltpu.VMEM((CHUNK, DH), jnp.float32)] * NBUF
    + [
        pltpu.VMEM_SHARED((NPAD, DH), jnp.float32),
    ]
    + [pltpu.SemaphoreType.DMA] * (2 * NBUF),
    compiler_params=pltpu.CompilerParams(use_tc_tiling_on_sc=False),
)
def _sc_aggregate(x_hbm, src_hbm, dst_hbm, zeros_hbm, out_hbm,
                  src_v, dst_v, *rest):
    rows = list(rest[:NBUF])
    acc = rest[NBUF]
    gsems = list(rest[NBUF + 1:NBUF + 1 + NBUF])
    ssems = list(rest[NBUF + 1 + NBUF:])
    c = lax.axis_index("c")
    s = lax.axis_index("s")
    wrow = s * CPT

    pltpu.sync_copy(src_hbm.at[c, pl.ds(wrow, CPT), :], src_v)
    pltpu.sync_copy(dst_hbm.at[pl.ds(wrow, CPT)], dst_v)
    cps = [None] * NBUF
    for j in range(NBUF - 2):
        cps[j] = pltpu.async_copy(x_hbm.at[src_v.at[j]], rows[j], gsems[j])
    pltpu.sync_copy(zeros_hbm.at[pl.ds(s * ZR, ZR)], acc.at[pl.ds(s * ZR, ZR)])
    plsc.subcore_barrier()

    scps = [None] * NBUF
    for j in range(CPT):
        b = j % NBUF
        cps[b].wait()
        scps[b] = pltpu.async_copy(rows[b], acc.at[dst_v.at[j]], ssems[b],
                                   add=True)
        g = j + NBUF - 2
        if g < CPT:
            bg = g % NBUF
            if scps[bg] is not None:
                scps[bg].wait()
                scps[bg] = None
            cps[bg] = pltpu.async_copy(x_hbm.at[src_v.at[g]], rows[bg],
                                       gsems[bg])
    for b in range(NBUF):
        if scps[b] is not None:
            scps[b].wait()
    plsc.subcore_barrier()
    pltpu.sync_copy(acc.at[pl.ds(s * ZR, ZR)], out_hbm.at[c, pl.ds(s * ZR, ZR), :])


def _mlp_body(x_r, p_r, wa_r, ba_r, wb_r, bb_r, o_r):
    pr = p_r[...]
    h = x_r[...] + jnp.concatenate([pr[0], pr[1]], axis=1)
    t = jnp.dot(h, wa_r[...], preferred_element_type=jnp.float32) + ba_r[...]
    t = jnp.maximum(t, 0.0)
    o = jnp.dot(t, wb_r[...], preferred_element_type=jnp.float32) + bb_r[...]
    o_r[...] = jnp.maximum(o, 0.0)


_BR = 1000


def _mlp(x, p, Wa, ba, Wb, bb):
    return pl.pallas_call(
        _mlp_body,
        grid=(N // _BR,),
        in_specs=[
            pl.BlockSpec((_BR, D), lambda i: (i, 0)),
            pl.BlockSpec((NC, _BR, DH), lambda i: (0, i, 0)),
            pl.BlockSpec((D, D), lambda i: (0, 0)),
            pl.BlockSpec((1, D), lambda i: (0, 0)),
            pl.BlockSpec((D, D), lambda i: (0, 0)),
            pl.BlockSpec((1, D), lambda i: (0, 0)),
        ],
        out_specs=pl.BlockSpec((_BR, D), lambda i: (i, 0)),
        out_shape=jax.ShapeDtypeStruct((N, D), jnp.float32),
    )(x, p, Wa, ba, Wb, bb)


def kernel(x, edge_index, W1a, b1a, W1b, b1b, W2a, b2a, W2b, b2b):
    src = edge_index[0].astype(jnp.int32)
    dst = edge_index[1].astype(jnp.int32)
    padt = EPT - E // NS
    pad_src = jnp.broadcast_to((jnp.arange(padt, dtype=jnp.int32) * 41) % N,
                               (NS, padt))
    pad_dst = jnp.broadcast_to(N + jnp.arange(padt, dtype=jnp.int32) % (NPAD - N),
                               (NS, padt))
    src_p = jnp.concatenate([src.reshape(NS, E // NS), pad_src], axis=1) * 2
    src_p = src_p.reshape(1, E_PAD // CHUNK, CHUNK)
    src_p = jnp.concatenate([src_p, src_p + 1], axis=0)
    dst_p = jnp.concatenate([dst.reshape(NS, E // NS), pad_dst], axis=1)
    dst_p = dst_p.reshape(E_PAD // CHUNK, CHUNK)
    zeros = jnp.zeros((NPAD, DH), jnp.float32)

    b1a2, b1b2 = b1a.reshape(1, D), b1b.reshape(1, D)
    b2a2, b2b2 = b2a.reshape(1, D), b2b.reshape(1, D)

    x2 = x.reshape(2 * N, DH)
    p = _sc_aggregate(x2, src_p, dst_p, zeros)
    h1 = _mlp(x, p, W1a, b1a2, W1b, b1b2)
    q = _sc_aggregate(h1.reshape(2 * N, DH), src_p, dst_p, zeros)
    h2 = _mlp(h1, q, W2a, b2a2, W2b, b2b2)
    return h2
